# Initial kernel scaffold; baseline (speedup 1.0000x reference)
import jax
import jax.numpy as jnp
from jax.experimental import pallas as pl


def kernel(inputs):
    raise NotImplementedError("write your pallas kernel here")


# speedup vs baseline: 20.8445x; 20.8445x over previous
import functools

import jax
import jax.numpy as jnp
from jax import lax
from jax.experimental import pallas as pl
from jax.experimental.pallas import tpu as pltpu

TOPK_N = 16
NEG = -3.0e38
BIG = 1e9


def _topk_block_kernel(x_ref, out_ref, *, n_cols):
    x = x_ref[...]
    rows = x.shape[0]
    iota = lax.broadcasted_iota(jnp.int32, (rows, n_cols), 1).astype(jnp.float32)
    vals = x
    vs = []
    js = []
    for _ in range(TOPK_N):
        m = jnp.max(vals, axis=1, keepdims=True)
        cand = jnp.where(vals == m, iota, BIG)
        j = jnp.min(cand, axis=1, keepdims=True)
        vs.append(m)
        js.append(j)
        vals = jnp.where(iota == j, NEG, vals)
    v0, v1, v2 = vs[0], vs[1], vs[2]
    j0, j1, j2 = js[0], js[1], js[2]
    top3_mean = (j0 + j1 + j2) * jnp.float32(1.0 / 3.0)
    top3_rms = jnp.sqrt((v0 * v0 + v1 * v1 + v2 * v2) * jnp.float32(1.0 / 3.0))
    max_rms = jnp.abs(v0)
    out = jnp.concatenate([top3_mean, top3_rms, j0, max_rms] + js, axis=1)
    out_ref[...] = out


def kernel(inputs):
    n_rows = inputs.shape[1] // 2
    x = inputs[:n_rows]
    n_cols = x.shape[1]
    block_rows = min(256, n_rows)
    grid = (n_rows // block_rows,)
    out = pl.pallas_call(
        functools.partial(_topk_block_kernel, n_cols=n_cols),
        grid=grid,
        in_specs=[pl.BlockSpec((block_rows, n_cols), lambda i: (i, 0))],
        out_specs=pl.BlockSpec((block_rows, TOPK_N + 4), lambda i: (i, 0)),
        out_shape=jax.ShapeDtypeStruct((n_rows, TOPK_N + 4), jnp.float32),
    )(x)
    return out


# speedup vs baseline: 56.4372x; 2.7075x over previous
import functools

import jax
import jax.numpy as jnp
from jax import lax
from jax.experimental import pallas as pl
from jax.experimental.pallas import tpu as pltpu
from jax.experimental.pallas import tpu_sc as plsc

TOPK_N = 16
NEG = -3.0e38
BIG = 1e9
N_COLS = 8192
N_GROUPS = 512
N_SLABS = 16


def _select_groups_kernel(x_ref, wg_ref):
    x = x_ref[...]
    rows = x.shape[0]
    gm = x[:, 0:N_GROUPS]
    for s in range(1, N_SLABS):
        gm = jnp.maximum(gm, x[:, s * N_GROUPS:(s + 1) * N_GROUPS])
    iota = lax.broadcasted_iota(jnp.int32, (rows, N_GROUPS), 1).astype(jnp.float32)
    js = []
    for _ in range(TOPK_N):
        m = jnp.max(gm, axis=1, keepdims=True)
        cand = jnp.where(gm == m, iota, BIG)
        j = jnp.min(cand, axis=1, keepdims=True)
        js.append(j)
        gm = jnp.where(iota == j, NEG, gm)
    wg_ref[...] = jnp.concatenate(js, axis=1).astype(jnp.int32)


def _vsqrt(a):
    ai = plsc.bitcast(a, jnp.int32)
    y = plsc.bitcast((ai >> 1) + 0x1FBD1DF5, jnp.float32)
    for _ in range(3):
        y = 0.5 * (y + a / y)
    return y


def _bvec(s):
    return lax.broadcast_in_dim(s, (16,), ())


def _sc_row(r, xbuf, wgv, outv, lane):
    wrow = wgv[r, :]
    rv = None
    ri = None
    for s in range(N_SLABS):
        idxv = wrow + s * N_GROUPS
        c = plsc.load_gather(xbuf, [idxv])
        cs, cis = plsc.sort_key_val(c, idxv, descending=True)
        if rv is None:
            rv, ri = cs, cis
        else:
            rc = lax.rev(cs, (0,))
            rci = lax.rev(cis, (0,))
            take = rv >= rc
            lv = jnp.where(take, rv, rc)
            li = jnp.where(take, ri, rci)
            rv, ri = plsc.sort_key_val(lv, li, descending=True)
    rif = ri.astype(jnp.float32)
    m3 = lane < 3
    m0 = lane == 0
    top3_mean = _bvec(jnp.sum(jnp.where(m3, rif, 0.0))) * (1.0 / 3.0)
    top3_rms = _vsqrt(_bvec(jnp.sum(jnp.where(m3, rv * rv, 0.0))) * (1.0 / 3.0))
    j0 = _bvec(jnp.sum(jnp.where(m0, rif, 0.0)))
    max_rms = jnp.abs(_bvec(jnp.sum(jnp.where(m0, rv, 0.0))))
    sv = jnp.where(lane == 0, top3_mean,
                   jnp.where(lane == 1, top3_rms,
                             jnp.where(lane == 2, j0, max_rms)))
    outv[r, pl.ds(0, 16)] = sv
    outv[r, pl.ds(16, 16)] = rif


def _sc_topk(x, wg, n_rows):
    try:
        info = plsc.get_sparse_core_info()
        nc, ns = info.num_cores, info.num_subcores
    except Exception:
        nc, ns = 2, 16
    nw = nc * ns
    rows_per_w = n_rows // nw
    n_pairs = rows_per_w // 2
    mesh = plsc.VectorSubcoreMesh(
        core_axis_name="c", subcore_axis_name="s",
        num_cores=nc, num_subcores=ns)

    @functools.partial(
        pl.kernel,
        mesh=mesh,
        out_type=jax.ShapeDtypeStruct((n_rows, 32), jnp.float32),
        compiler_params=pltpu.CompilerParams(needs_layout_passes=False),
        scratch_types=[
            pltpu.VMEM((N_COLS,), jnp.float32),
            pltpu.VMEM((N_COLS,), jnp.float32),
            pltpu.VMEM((rows_per_w, TOPK_N), jnp.int32),
            pltpu.VMEM((rows_per_w, 32), jnp.float32),
            pltpu.SemaphoreType.DMA,
            pltpu.SemaphoreType.DMA,
        ],
    )
    def sc_kernel(x_hbm, wg_hbm, out_hbm, xbuf0, xbuf1, wgv, outv,
                  sem0, sem1):
        wid = lax.axis_index("s") * nc + lax.axis_index("c")
        base = wid * rows_per_w
        lane = lax.iota(jnp.int32, 16)
        pltpu.sync_copy(wg_hbm.at[pl.ds(base, rows_per_w), :], wgv)
        pltpu.async_copy(x_hbm.at[base], xbuf0, sem0)

        def pair_body(p, carry):
            r0 = 2 * p
            pltpu.async_copy(x_hbm.at[base + r0 + 1], xbuf1, sem1)
            pltpu.make_async_copy(x_hbm.at[base + r0], xbuf0, sem0).wait()
            _sc_row(r0, xbuf0, wgv, outv, lane)

            @pl.when(p < n_pairs - 1)
            def _():
                pltpu.async_copy(x_hbm.at[base + r0 + 2], xbuf0, sem0)

            pltpu.make_async_copy(x_hbm.at[base + r0 + 1], xbuf1, sem1).wait()
            _sc_row(r0 + 1, xbuf1, wgv, outv, lane)
            return carry

        lax.fori_loop(0, n_pairs, pair_body, 0)
        pltpu.sync_copy(outv, out_hbm.at[pl.ds(base, rows_per_w), :])

    return sc_kernel(x, wg)


def kernel(inputs):
    n_rows = inputs.shape[1] // 2
    x = inputs[:n_rows]
    block_rows = 256
    wg = pl.pallas_call(
        _select_groups_kernel,
        grid=(n_rows // block_rows,),
        in_specs=[pl.BlockSpec((block_rows, N_COLS), lambda i: (i, 0))],
        out_specs=pl.BlockSpec((block_rows, TOPK_N), lambda i: (i, 0)),
        out_shape=jax.ShapeDtypeStruct((n_rows, TOPK_N), jnp.int32),
    )(x)
    out32 = _sc_topk(x, wg, n_rows)
    return jnp.concatenate([out32[:, :4], out32[:, 16:32]], axis=1)


# speedup vs baseline: 57.1333x; 1.0123x over previous
import functools

import jax
import jax.numpy as jnp
from jax import lax
from jax.experimental import pallas as pl
from jax.experimental.pallas import tpu as pltpu
from jax.experimental.pallas import tpu_sc as plsc

TOPK_N = 16
NEG = -3.0e38
BIG = 1e9
N_COLS = 8192
N_GROUPS = 512
N_SLABS = 16


def _select_groups_kernel(x_ref, wg_ref):
    x = x_ref[...]
    rows = x.shape[0]
    gm = x[:, 0:N_GROUPS]
    for s in range(1, N_SLABS):
        gm = jnp.maximum(gm, x[:, s * N_GROUPS:(s + 1) * N_GROUPS])
    iota = lax.broadcasted_iota(jnp.int32, (rows, N_GROUPS), 1).astype(jnp.float32)
    js = []
    for _ in range(TOPK_N):
        m = jnp.max(gm, axis=1, keepdims=True)
        cand = jnp.where(gm == m, iota, BIG)
        j = jnp.min(cand, axis=1, keepdims=True)
        js.append(j)
        gm = jnp.where(iota == j, NEG, gm)
    wg_ref[...] = jnp.concatenate(js, axis=1).astype(jnp.int32)


def _vsqrt(a):
    ai = plsc.bitcast(a, jnp.int32)
    y = plsc.bitcast((ai >> 1) + 0x1FBD1DF5, jnp.float32)
    for _ in range(3):
        y = 0.5 * (y + a / y)
    return y


def _bvec(s):
    return lax.broadcast_in_dim(s, (16,), ())


def _lex_merge(a, b):
    av, ai = a
    rb = lax.rev(b[0], (0,))
    rbi = lax.rev(b[1], (0,))
    take = (av > rb) | ((av == rb) & (ai < rbi))
    lv = jnp.where(take, av, rb)
    li = jnp.where(take, ai, rbi)
    return plsc.sort_key_val(lv, li, descending=True)


def _tie_fix_pass(rv, ri, lane, perm):
    _, pv = plsc.sort_key_val(perm, rv)
    _, pi = plsc.sort_key_val(perm, ri)
    eq = rv == pv
    first = lane < perm
    return jnp.where(eq, jnp.where(first, jnp.minimum(ri, pi),
                                   jnp.maximum(ri, pi)), ri)


def _sc_row(r, xbuf, wgv, outv, lane):
    wrow = wgv[r, :]
    level = []
    for s in range(N_SLABS):
        idxv = wrow + s * N_GROUPS
        c = plsc.load_gather(xbuf, [idxv])
        level.append(plsc.sort_key_val(c, idxv, descending=True))
    while len(level) > 1:
        level = [_lex_merge(level[i], level[i + 1])
                 for i in range(0, len(level), 2)]
    rv, ri = level[0]
    odd = (lane & 1) == 1
    perm1 = lane ^ 1
    perm2 = jnp.where((lane >= 1) & (lane <= 14),
                      jnp.where(odd, lane + 1, lane - 1), lane)
    ri = _tie_fix_pass(rv, ri, lane, perm1)
    ri = _tie_fix_pass(rv, ri, lane, perm2)
    ri = _tie_fix_pass(rv, ri, lane, perm1)
    rif = ri.astype(jnp.float32)
    m3 = lane < 3
    m0 = lane == 0
    top3_mean = _bvec(jnp.sum(jnp.where(m3, rif, 0.0))) * (1.0 / 3.0)
    top3_rms = _vsqrt(_bvec(jnp.sum(jnp.where(m3, rv * rv, 0.0))) * (1.0 / 3.0))
    j0 = _bvec(jnp.sum(jnp.where(m0, rif, 0.0)))
    max_rms = jnp.abs(_bvec(jnp.sum(jnp.where(m0, rv, 0.0))))
    sv = jnp.where(lane == 0, top3_mean,
                   jnp.where(lane == 1, top3_rms,
                             jnp.where(lane == 2, j0, max_rms)))
    outv[r, pl.ds(0, 16)] = sv
    outv[r, pl.ds(16, 16)] = rif


def _sc_topk(x, wg, n_rows):
    try:
        info = plsc.get_sparse_core_info()
        nc, ns = info.num_cores, info.num_subcores
    except Exception:
        nc, ns = 2, 16
    nw = nc * ns
    rows_per_w = n_rows // nw
    n_pairs = rows_per_w // 2
    mesh = plsc.VectorSubcoreMesh(
        core_axis_name="c", subcore_axis_name="s",
        num_cores=nc, num_subcores=ns)

    @functools.partial(
        pl.kernel,
        mesh=mesh,
        out_type=jax.ShapeDtypeStruct((n_rows, 32), jnp.float32),
        compiler_params=pltpu.CompilerParams(needs_layout_passes=False),
        scratch_types=[
            pltpu.VMEM((N_COLS,), jnp.float32),
            pltpu.VMEM((N_COLS,), jnp.float32),
            pltpu.VMEM((rows_per_w, TOPK_N), jnp.int32),
            pltpu.VMEM((rows_per_w, 32), jnp.float32),
            pltpu.SemaphoreType.DMA,
            pltpu.SemaphoreType.DMA,
        ],
    )
    def sc_kernel(x_hbm, wg_hbm, out_hbm, xbuf0, xbuf1, wgv, outv,
                  sem0, sem1):
        wid = lax.axis_index("s") * nc + lax.axis_index("c")
        base = wid * rows_per_w
        lane = lax.iota(jnp.int32, 16)
        pltpu.sync_copy(wg_hbm.at[pl.ds(base, rows_per_w), :], wgv)
        pltpu.async_copy(x_hbm.at[base], xbuf0, sem0)

        def pair_body(p, carry):
            r0 = 2 * p
            pltpu.async_copy(x_hbm.at[base + r0 + 1], xbuf1, sem1)
            pltpu.make_async_copy(x_hbm.at[base + r0], xbuf0, sem0).wait()
            _sc_row(r0, xbuf0, wgv, outv, lane)

            @pl.when(p < n_pairs - 1)
            def _():
                pltpu.async_copy(x_hbm.at[base + r0 + 2], xbuf0, sem0)

            pltpu.make_async_copy(x_hbm.at[base + r0 + 1], xbuf1, sem1).wait()
            _sc_row(r0 + 1, xbuf1, wgv, outv, lane)
            return carry

        lax.fori_loop(0, n_pairs, pair_body, 0)
        pltpu.sync_copy(outv, out_hbm.at[pl.ds(base, rows_per_w), :])

    return sc_kernel(x, wg)


def kernel(inputs):
    n_rows = inputs.shape[1] // 2
    x = inputs[:n_rows]
    block_rows = 256
    wg = pl.pallas_call(
        _select_groups_kernel,
        grid=(n_rows // block_rows,),
        in_specs=[pl.BlockSpec((block_rows, N_COLS), lambda i: (i, 0))],
        out_specs=pl.BlockSpec((block_rows, TOPK_N), lambda i: (i, 0)),
        out_shape=jax.ShapeDtypeStruct((n_rows, TOPK_N), jnp.int32),
    )(x)
    out32 = _sc_topk(x, wg, n_rows)
    return jnp.concatenate([out32[:, :4], out32[:, 16:32]], axis=1)
